# baseline (speedup 1.0000x reference)
import functools
import numpy as np

import jax
import jax.numpy as jnp
from jax.experimental import pallas as pl

DIM = 768
GRID_SIZE = 64
NUM_ROUTES = 8
TEMP = 0.1
SPARSITY = 0.3
QBLK = 256


def _mm_nt(a, b):
    return jax.lax.dot_general(
        a, b, (((1,), (1,)), ((), ())), preferred_element_type=jnp.float32)


def _mm_nn(a, b):
    return jax.lax.dot_general(
        a, b, (((1,), (0,)), ((), ())), preferred_element_type=jnp.float32)


def _proj_body(x_ref, wq_ref, bq_ref, wk_ref, bk_ref, wv_ref, bv_ref,
               fp_ref, wg_ref, bg_ref, qn_ref, kn_ref, vg_ref):
    x = x_ref[0]
    q = _mm_nt(x, wq_ref[...]) + bq_ref[...]
    k = _mm_nt(x, wk_ref[...]) + bk_ref[...]
    v = _mm_nt(x, wv_ref[...]) + bv_ref[...]
    qn = jnp.sqrt(jnp.sum(q * q, axis=-1, keepdims=True))
    kn = jnp.sqrt(jnp.sum(k * k, axis=-1, keepdims=True))
    qn_ref[0] = q / jnp.maximum(qn, 1e-12)
    kn_ref[0] = k / jnp.maximum(kn, 1e-12)
    gate = jax.nn.sigmoid(
        jnp.sum(fp_ref[...] * wg_ref[...], axis=1)[None, :] + bg_ref[...])
    vg_ref[0] = v * gate


def _route_body(qn_ref, kn_ref, vg_ref, x_ref, wo_ref, bo_ref,
                cn_ref, thr_ref, routes_ref, weights_ref, out_ref,
                *, n_pad, p_valid):
    qb = pl.program_id(1)
    row0 = qb * QBLK
    q = qn_ref[0]
    kk = kn_ref[0]
    s = _mm_nt(q, kk)

    cn_cols = cn_ref[0, 0]
    cn_rows = cn_ref[0, 0, pl.ds(row0, QBLK)]
    thr_rows = thr_ref[0, 0, pl.ds(row0, QBLK)]

    aff = 1.0 - jnp.abs(cn_cols[None, :] - cn_rows[:, None])
    col = jax.lax.broadcasted_iota(jnp.int32, (QBLK, n_pad), 1)
    row_g = jax.lax.broadcasted_iota(jnp.int32, (QBLK, n_pad), 0) + row0
    mask = (aff < thr_rows[:, None]) | (col == row_g) | (col >= p_valid)
    s = jnp.where(mask, -1e9, s)
    st = s / TEMP

    vals = []
    idxs = []
    for _ in range(NUM_ROUTES):
        m = jnp.max(st, axis=1)
        is_max = st == m[:, None]
        idx = jnp.min(jnp.where(is_max, col, n_pad), axis=1)
        vals.append(m)
        idxs.append(idx)
        st = jnp.where(col == idx[:, None], -3e38, st)
    topv = jnp.stack(vals, axis=1)
    topi = jnp.stack(idxs, axis=1)

    mx = jnp.max(topv, axis=1, keepdims=True)
    un = jnp.exp(topv - mx)
    w = un / jnp.sum(un, axis=1, keepdims=True)

    routes_ref[0] = topi
    weights_ref[0] = w

    a = jnp.zeros((QBLK, n_pad), jnp.float32)
    for t in range(NUM_ROUTES):
        a = a + jnp.where(col == topi[:, t][:, None], w[:, t][:, None], 0.0)
    routed = _mm_nn(a, vg_ref[0])
    out_ref[0] = x_ref[0] + _mm_nt(routed, wo_ref[...]) + bo_ref[...]


def _cantor_thresholds(p_valid, n_pad, fp, W_off, b_off):
    pos = np.arange(n_pad)
    xg = pos % GRID_SIZE
    yg = pos // GRID_SIZE
    base = (((xg + yg) * (xg + yg + 1)) // 2 + yg).astype(np.int64)
    k_sparse = max(1, int(p_valid * SPARSITY))

    params = fp @ W_off.T + b_off
    scale = jax.nn.sigmoid(params[0]) * 2 + 0.5
    shift = jnp.tanh(params[1:2]) * GRID_SIZE
    base_f = jnp.asarray(base, jnp.float32)
    offset = base_f * scale + jnp.sum(shift)
    cn = offset / jnp.maximum(jnp.max(offset), 1.0)

    perm = np.argsort(base[:p_valid], kind="stable")
    inv = np.empty(p_valid, dtype=np.int64)
    inv[perm] = np.arange(p_valid)

    s = cn[perm]
    t = np.arange(k_sparse + 1)
    r = np.arange(p_valid)
    l_idx = r[None, :] - t[:, None]
    valid = (l_idx >= 0) & (l_idx + k_sparse <= p_valid - 1)
    l_idx_c = np.clip(l_idx, 0, p_valid - 1 - k_sparse)
    sl = s[l_idx_c]
    sr = s[l_idx_c + k_sparse]
    cand = jnp.maximum(s[None, :] - sl, sr - s[None, :])
    cand = jnp.where(jnp.asarray(valid), cand, jnp.inf)
    dk = jnp.min(cand, axis=0)
    thr_sorted = 1.0 - dk
    thr = thr_sorted[inv]
    thr_pad = jnp.concatenate(
        [thr, jnp.full((n_pad - p_valid,), jnp.inf, jnp.float32)])
    return cn, thr_pad


def kernel(x, Wq, bq, Wk, bk, Wv, bv, Wo, bo, Wg, bg, fp, W_off, b_off):
    B = x.shape[0]
    d = x.shape[-1]
    cls_token = x[:, :1, :]
    xb = x[:, 1:, :]
    p_valid = xb.shape[1]
    n_pad = ((p_valid + QBLK - 1) // QBLK) * QBLK
    nqb = n_pad // QBLK

    xp = jnp.pad(xb, ((0, 0), (0, n_pad - p_valid), (0, 0)))
    cn, thr = _cantor_thresholds(p_valid, n_pad, fp, W_off, b_off)
    cn3 = cn.reshape(1, 1, n_pad)
    thr3 = thr.reshape(1, 1, n_pad)

    bq2, bk2, bv2, bo2 = (b.reshape(1, d) for b in (bq, bk, bv, bo))
    bg2 = bg.reshape(1, d)
    fp2 = fp.reshape(1, -1)

    row_blk = pl.BlockSpec((1, QBLK, d), lambda b, i: (b, i, 0))
    full_mat = pl.BlockSpec((d, d), lambda b, i: (0, 0))
    full_bias = pl.BlockSpec((1, d), lambda b, i: (0, 0))

    qn, kn, vg = pl.pallas_call(
        _proj_body,
        grid=(B, nqb),
        in_specs=[
            row_blk,
            full_mat, full_bias,
            full_mat, full_bias,
            full_mat, full_bias,
            pl.BlockSpec((1, fp.shape[0]), lambda b, i: (0, 0)),
            pl.BlockSpec((d, fp.shape[0]), lambda b, i: (0, 0)),
            full_bias,
        ],
        out_specs=[row_blk, row_blk, row_blk],
        out_shape=[jax.ShapeDtypeStruct((B, n_pad, d), jnp.float32)] * 3,
    )(xp, Wq, bq2, Wk, bk2, Wv, bv2, fp2, Wg, bg2)

    batch_full = pl.BlockSpec((1, n_pad, d), lambda b, i: (b, 0, 0))
    vec_full = pl.BlockSpec((1, 1, n_pad), lambda b, i: (0, 0, 0))

    routes, weights, out = pl.pallas_call(
        functools.partial(_route_body, n_pad=n_pad, p_valid=p_valid),
        grid=(B, nqb),
        in_specs=[
            row_blk,
            batch_full,
            batch_full,
            row_blk,
            full_mat,
            full_bias,
            vec_full,
            vec_full,
        ],
        out_specs=[
            pl.BlockSpec((1, QBLK, NUM_ROUTES), lambda b, i: (b, i, 0)),
            pl.BlockSpec((1, QBLK, NUM_ROUTES), lambda b, i: (b, i, 0)),
            row_blk,
        ],
        out_shape=[
            jax.ShapeDtypeStruct((B, n_pad, NUM_ROUTES), jnp.int32),
            jax.ShapeDtypeStruct((B, n_pad, NUM_ROUTES), jnp.float32),
            jax.ShapeDtypeStruct((B, n_pad, d), jnp.float32),
        ],
    )(qn, kn, vg, xp, Wo, bo2, cn3, thr3)

    routes = routes[:, :p_valid]
    weights = weights[:, :p_valid]
    output = jnp.concatenate([cls_token, out[:, :p_valid]], axis=1)
    return (routes, weights, output)


# speedup vs baseline: 68.1821x; 68.1821x over previous
import functools
import numpy as np

import jax
import jax.numpy as jnp
from jax.experimental import pallas as pl

DIM = 768
GRID_SIZE = 64
NUM_ROUTES = 8
TEMP = 0.1
SPARSITY = 0.3
QBLK = 256


def _mm_nt(a, b):
    return jax.lax.dot_general(
        a, b, (((1,), (1,)), ((), ())), preferred_element_type=jnp.float32)


def _mm_nn(a, b):
    return jax.lax.dot_general(
        a, b, (((1,), (0,)), ((), ())), preferred_element_type=jnp.float32)


def _proj_body(x_ref, wq_ref, bq_ref, wk_ref, bk_ref, wv_ref, bv_ref,
               fp_ref, wg_ref, bg_ref, qn_ref, kn_ref, vg_ref):
    x = x_ref[0]
    q = _mm_nt(x, wq_ref[...]) + bq_ref[...]
    k = _mm_nt(x, wk_ref[...]) + bk_ref[...]
    v = _mm_nt(x, wv_ref[...]) + bv_ref[...]
    qn = jnp.sqrt(jnp.sum(q * q, axis=-1, keepdims=True))
    kn = jnp.sqrt(jnp.sum(k * k, axis=-1, keepdims=True))
    qn_ref[0] = q / jnp.maximum(qn, 1e-12)
    kn_ref[0] = k / jnp.maximum(kn, 1e-12)
    gate = jax.nn.sigmoid(
        jnp.sum(fp_ref[...] * wg_ref[...], axis=1)[None, :] + bg_ref[...])
    vg_ref[0] = v * gate


def _route_body(qn_ref, kn_ref, vg_ref, x_ref, wo_ref, bo_ref,
                cn_ref, thr_ref, routes_ref, weights_ref, out_ref,
                *, n_pad, p_valid):
    qb = pl.program_id(1)
    row0 = qb * QBLK
    q = qn_ref[0]
    kk = kn_ref[0]
    s = _mm_nt(q, kk)

    cn_cols = cn_ref[0, 0]
    cn_rows = cn_ref[0, 0, pl.ds(row0, QBLK)]
    thr_rows = thr_ref[0, 0, pl.ds(row0, QBLK)]

    aff = 1.0 - jnp.abs(cn_cols[None, :] - cn_rows[:, None])
    col = jax.lax.broadcasted_iota(jnp.int32, (QBLK, n_pad), 1)
    row_g = jax.lax.broadcasted_iota(jnp.int32, (QBLK, n_pad), 0) + row0
    mask = (aff < thr_rows[:, None]) | (col == row_g) | (col >= p_valid)
    s = jnp.where(mask, -1e9, s)
    st = s / TEMP

    vals = []
    idxs = []
    for _ in range(NUM_ROUTES):
        m = jnp.max(st, axis=1)
        is_max = st == m[:, None]
        idx = jnp.min(jnp.where(is_max, col, n_pad), axis=1)
        vals.append(m)
        idxs.append(idx)
        st = jnp.where(col == idx[:, None], -3e38, st)
    topv = jnp.stack(vals, axis=1)
    topi = jnp.stack(idxs, axis=1)

    mx = jnp.max(topv, axis=1, keepdims=True)
    un = jnp.exp(topv - mx)
    w = un / jnp.sum(un, axis=1, keepdims=True)

    routes_ref[0] = topi
    weights_ref[0] = w

    a = jnp.zeros((QBLK, n_pad), jnp.float32)
    for t in range(NUM_ROUTES):
        a = a + jnp.where(col == topi[:, t][:, None], w[:, t][:, None], 0.0)
    routed = _mm_nn(a, vg_ref[0])
    out_ref[0] = x_ref[0] + _mm_nt(routed, wo_ref[...]) + bo_ref[...]


def _cantor_thresholds(p_valid, n_pad, fp, W_off, b_off):
    pos = np.arange(n_pad)
    xg = pos % GRID_SIZE
    yg = pos // GRID_SIZE
    base = (((xg + yg) * (xg + yg + 1)) // 2 + yg).astype(np.int64)
    k_sparse = max(1, int(p_valid * SPARSITY))

    params = fp @ W_off.T + b_off
    scale = jax.nn.sigmoid(params[0]) * 2 + 0.5
    shift = jnp.tanh(params[1:2]) * GRID_SIZE
    base_f = jnp.asarray(base, jnp.float32)
    offset = base_f * scale + jnp.sum(shift)
    cn = offset / jnp.maximum(jnp.max(offset), 1.0)

    perm = np.argsort(base[:p_valid], kind="stable")
    inv = np.empty(p_valid, dtype=np.int64)
    inv[perm] = np.arange(p_valid)

    s = cn[perm]
    neg = jnp.full((k_sparse,), -jnp.inf, jnp.float32)
    pos_inf = jnp.full((k_sparse,), jnp.inf, jnp.float32)
    s_ext = jnp.concatenate([neg, s, pos_inf])
    dk = jnp.full((p_valid,), jnp.inf, jnp.float32)
    for t in range(k_sparse + 1):
        sl = jax.lax.slice(s_ext, (k_sparse - t,), (k_sparse - t + p_valid,))
        sr = jax.lax.slice(s_ext, (2 * k_sparse - t,),
                           (2 * k_sparse - t + p_valid,))
        dk = jnp.minimum(dk, jnp.maximum(s - sl, sr - s))
    thr_sorted = 1.0 - dk
    thr = thr_sorted[inv]
    thr_pad = jnp.concatenate(
        [thr, jnp.full((n_pad - p_valid,), jnp.inf, jnp.float32)])
    return cn, thr_pad


def kernel(x, Wq, bq, Wk, bk, Wv, bv, Wo, bo, Wg, bg, fp, W_off, b_off):
    B = x.shape[0]
    d = x.shape[-1]
    cls_token = x[:, :1, :]
    xb = x[:, 1:, :]
    p_valid = xb.shape[1]
    n_pad = ((p_valid + QBLK - 1) // QBLK) * QBLK
    nqb = n_pad // QBLK

    xp = jnp.pad(xb, ((0, 0), (0, n_pad - p_valid), (0, 0)))
    cn, thr = _cantor_thresholds(p_valid, n_pad, fp, W_off, b_off)
    cn3 = cn.reshape(1, 1, n_pad)
    thr3 = thr.reshape(1, 1, n_pad)

    bq2, bk2, bv2, bo2 = (b.reshape(1, d) for b in (bq, bk, bv, bo))
    bg2 = bg.reshape(1, d)
    fp2 = fp.reshape(1, -1)

    row_blk = pl.BlockSpec((1, QBLK, d), lambda b, i: (b, i, 0))
    full_mat = pl.BlockSpec((d, d), lambda b, i: (0, 0))
    full_bias = pl.BlockSpec((1, d), lambda b, i: (0, 0))

    qn, kn, vg = pl.pallas_call(
        _proj_body,
        grid=(B, nqb),
        in_specs=[
            row_blk,
            full_mat, full_bias,
            full_mat, full_bias,
            full_mat, full_bias,
            pl.BlockSpec((1, fp.shape[0]), lambda b, i: (0, 0)),
            pl.BlockSpec((d, fp.shape[0]), lambda b, i: (0, 0)),
            full_bias,
        ],
        out_specs=[row_blk, row_blk, row_blk],
        out_shape=[jax.ShapeDtypeStruct((B, n_pad, d), jnp.float32)] * 3,
    )(xp, Wq, bq2, Wk, bk2, Wv, bv2, fp2, Wg, bg2)

    batch_full = pl.BlockSpec((1, n_pad, d), lambda b, i: (b, 0, 0))
    vec_full = pl.BlockSpec((1, 1, n_pad), lambda b, i: (0, 0, 0))

    routes, weights, out = pl.pallas_call(
        functools.partial(_route_body, n_pad=n_pad, p_valid=p_valid),
        grid=(B, nqb),
        in_specs=[
            row_blk,
            batch_full,
            batch_full,
            row_blk,
            full_mat,
            full_bias,
            vec_full,
            vec_full,
        ],
        out_specs=[
            pl.BlockSpec((1, QBLK, NUM_ROUTES), lambda b, i: (b, i, 0)),
            pl.BlockSpec((1, QBLK, NUM_ROUTES), lambda b, i: (b, i, 0)),
            row_blk,
        ],
        out_shape=[
            jax.ShapeDtypeStruct((B, n_pad, NUM_ROUTES), jnp.int32),
            jax.ShapeDtypeStruct((B, n_pad, NUM_ROUTES), jnp.float32),
            jax.ShapeDtypeStruct((B, n_pad, d), jnp.float32),
        ],
    )(qn, kn, vg, xp, Wo, bo2, cn3, thr3)

    routes = routes[:, :p_valid]
    weights = weights[:, :p_valid]
    output = jnp.concatenate([cls_token, out[:, :p_valid]], axis=1)
    return (routes, weights, output)
